# baseline (speedup 1.0000x reference)
import functools

import jax
import jax.numpy as jnp
from jax import lax
from jax.experimental import pallas as pl
from jax.experimental.pallas import tpu as pltpu
from jax.experimental.pallas import tpu_sc as plsc

CONT = 13
CATE = 26
NC = 2
NS = 16
NW = NC * NS


def _make_sc_kernel(B):
    bpw = B // NW
    n_idx = bpw * CATE
    mesh = plsc.VectorSubcoreMesh(core_axis_name="c", subcore_axis_name="s")

    @functools.partial(
        pl.kernel,
        out_type=jax.ShapeDtypeStruct((B,), jnp.float32),
        mesh=mesh,
        compiler_params=pltpu.CompilerParams(needs_layout_passes=False),
        scratch_types=[
            pltpu.VMEM((n_idx,), jnp.int32),
            pltpu.VMEM((n_idx,), jnp.float32),
            pltpu.VMEM((bpw * CONT,), jnp.float32),
            pltpu.VMEM((16,), jnp.float32),
            pltpu.VMEM((bpw,), jnp.float32),
            pltpu.SemaphoreType.DMA,
            pltpu.SemaphoreType.DMA,
            pltpu.SemaphoreType.DMA,
            pltpu.SemaphoreType.DMA,
        ],
    )
    def sc_kernel(table_hbm, idx_hbm, conts_hbm, out_hbm,
                  idx_v, gat_v, conts_v, w_v, out_v, s0, s1, s2, s3):
        wid = lax.axis_index("s") * NC + lax.axis_index("c")
        base = wid * bpw
        sems = [s0, s1, s2, s3]
        rows_per_seg = bpw // 4

        table_1d = table_hbm.at[0]

        def seg_copy(k):
            w0 = k * rows_per_seg * CATE
            nw_ = rows_per_seg * CATE
            return pltpu.make_async_copy(
                table_1d.at[idx_v.at[pl.ds(w0, nw_)]],
                gat_v.at[pl.ds(w0, nw_)],
                sems[k],
            )

        pltpu.sync_copy(idx_hbm.at[wid], idx_v)
        for k in range(4):
            seg_copy(k).start()
        pltpu.sync_copy(conts_hbm.at[wid], conts_v)
        pltpu.sync_copy(table_1d.at[pl.ds(0, 16)], w_v)

        wv = w_v[...]
        ws = [wv[j] for j in range(CONT)]
        lane = lax.iota(jnp.int32, 16)
        lane_cate = lane * CATE
        lane_cont = lane * CONT

        for k in range(4):
            seg_copy(k).wait()

            def body(g, carry, k=k):
                off = g * 16
                cbase = lane_cate + (k * rows_per_seg + off) * CATE
                fbase = lane_cont + (k * rows_per_seg + off) * CONT
                acc = plsc.load_gather(gat_v, [cbase])
                for j in range(1, CATE):
                    acc = acc + plsc.load_gather(gat_v, [cbase + j])
                for j in range(CONT):
                    acc = acc + plsc.load_gather(conts_v, [fbase + j]) * ws[j]
                out_v[pl.ds(k * rows_per_seg + off, 16)] = (
                    1.0 / (1.0 + jnp.exp(-acc)))
                return carry

            lax.fori_loop(0, rows_per_seg // 16, body, 0)

        pltpu.sync_copy(out_v, out_hbm.at[pl.ds(base, bpw)])

    return sc_kernel


def kernel(conts, cates, combs, table):
    del combs
    B = conts.shape[0]
    idx_r = cates.astype(jnp.int32).reshape(NW, (B // NW) * CATE)
    conts_r = conts.reshape(NW, (B // NW) * CONT)
    out = _make_sc_kernel(B)(table.T, idx_r, conts_r)
    return out.reshape(B, 1)


# speedup vs baseline: 1.5031x; 1.5031x over previous
import functools

import jax
import jax.numpy as jnp
from jax import lax
from jax.experimental import pallas as pl
from jax.experimental.pallas import tpu as pltpu
from jax.experimental.pallas import tpu_sc as plsc

CONT = 13
CATE = 26
NC = 2
NS = 16
NW = NC * NS


def _make_sc_kernel(B):
    bpw = B // NW
    n_idx = bpw * CATE
    mesh = plsc.VectorSubcoreMesh(core_axis_name="c", subcore_axis_name="s")

    @functools.partial(
        pl.kernel,
        out_type=jax.ShapeDtypeStruct((B,), jnp.float32),
        mesh=mesh,
        compiler_params=pltpu.CompilerParams(needs_layout_passes=False),
        scratch_types=[
            pltpu.VMEM((bpw * (CATE + CONT),), jnp.int32),
            pltpu.VMEM((n_idx,), jnp.float32),
            pltpu.VMEM((16,), jnp.float32),
            pltpu.VMEM((bpw,), jnp.float32),
            pltpu.SemaphoreType.DMA,
            pltpu.SemaphoreType.DMA,
            pltpu.SemaphoreType.DMA,
            pltpu.SemaphoreType.DMA,
        ],
    )
    def sc_kernel(table_hbm, packed_hbm, out_hbm,
                  pk_v, gat_v, w_v, out_v, s0, s1, s2, s3):
        wid = lax.axis_index("s") * NC + lax.axis_index("c")
        base = wid * bpw
        sems = [s0, s1, s2, s3]
        segs = [(0, 7), (7, 7), (14, 6), (20, 6)]

        table_1d = table_hbm.at[0]

        def seg_copy(k):
            j0, nj = segs[k]
            return pltpu.make_async_copy(
                table_1d.at[pk_v.at[pl.ds(j0 * bpw, nj * bpw)]],
                gat_v.at[pl.ds(j0 * bpw, nj * bpw)],
                sems[k],
            )

        pltpu.sync_copy(packed_hbm.at[wid], pk_v)
        for k in range(4):
            seg_copy(k).start()
        pltpu.sync_copy(table_1d.at[pl.ds(0, 16)], w_v)

        wv = w_v[...]
        ws = [wv[j] for j in range(CONT)]

        for k in range(4):
            j0, nj = segs[k]
            seg_copy(k).wait()

            def body(g, carry, j0=j0, nj=nj, k=k):
                off = g * 16
                if k == 0:
                    cv = plsc.bitcast(pk_v[pl.ds(CATE * bpw + off, 16)],
                                      jnp.float32)
                    acc = cv * ws[0]
                    for j in range(1, CONT):
                        cv = plsc.bitcast(
                            pk_v[pl.ds((CATE + j) * bpw + off, 16)],
                            jnp.float32)
                        acc = acc + cv * ws[j]
                else:
                    acc = out_v[pl.ds(off, 16)]
                for j in range(j0, j0 + nj):
                    acc = acc + gat_v[pl.ds(j * bpw + off, 16)]
                if k == 3:
                    acc = 1.0 / (1.0 + jnp.exp(-acc))
                out_v[pl.ds(off, 16)] = acc
                return carry

            lax.fori_loop(0, bpw // 16, body, 0)

        pltpu.sync_copy(out_v, out_hbm.at[pl.ds(base, bpw)])

    return sc_kernel


def kernel(conts, cates, combs, table):
    del combs
    B = conts.shape[0]
    bpw = B // NW
    packed = jnp.concatenate(
        [cates.astype(jnp.int32),
         jax.lax.bitcast_convert_type(conts, jnp.int32)], axis=1)
    packed_t = (
        packed.reshape(NW, bpw, CATE + CONT)
        .transpose(0, 2, 1)
        .reshape(NW, bpw * (CATE + CONT))
    )
    out = _make_sc_kernel(B)(table.T, packed_t)
    return out.reshape(B, 1)
